# baseline (speedup 1.0000x reference)
import jax
import jax.numpy as jnp
from jax.experimental import pallas as pl

_NUM_TAGS = 32
_SHIFT = 128
_L = _NUM_TAGS * _SHIFT
_D = 512
_B = 8
_K = 8
_LBLK = 256


def _copy_mask_kernel(tags_ref, emb_ref, out_ref, mask_ref):
    l = pl.program_id(0)
    x = emb_ref[...]
    out_ref[...] = jnp.broadcast_to(x[None], (_B, _LBLK, _D))
    base = l * _LBLK
    tag_ids = (base + jax.lax.broadcasted_iota(jnp.int32, (1, _LBLK), 1)) // _SHIFT
    tags = tags_ref[...]
    acc = jnp.zeros((_B, _LBLK), jnp.bool_)
    for k in range(_K):
        acc = acc | (tags[:, k : k + 1] == tag_ids)
    mask_ref[...] = acc.astype(jnp.int32)


def kernel(tags, embedding):
    num_l = _L // _LBLK
    embed, mask = pl.pallas_call(
        _copy_mask_kernel,
        grid=(num_l,),
        in_specs=[
            pl.BlockSpec((_B, _K), lambda l: (0, 0)),
            pl.BlockSpec((_LBLK, _D), lambda l: (l, 0)),
        ],
        out_specs=[
            pl.BlockSpec((_B, _LBLK, _D), lambda l: (0, l, 0)),
            pl.BlockSpec((_B, _LBLK), lambda l: (0, l)),
        ],
        out_shape=[
            jax.ShapeDtypeStruct((_B, _L, _D), jnp.float32),
            jax.ShapeDtypeStruct((_B, _L), jnp.int32),
        ],
    )(tags.astype(jnp.int32), embedding)
    return embed, mask


# speedup vs baseline: 1.0938x; 1.0938x over previous
import jax
import jax.numpy as jnp
from jax.experimental import pallas as pl

_NUM_TAGS = 32
_SHIFT = 128
_L = _NUM_TAGS * _SHIFT
_D = 512
_B = 8
_K = 8
_LBLK = 512


def _copy_mask_kernel(tags_ref, emb_ref, out_ref, mask_ref):
    l = pl.program_id(0)
    x = emb_ref[...]
    out_ref[...] = jnp.broadcast_to(x[None], (_B, _LBLK, _D))
    base = l * _LBLK
    tag_ids = (base + jax.lax.broadcasted_iota(jnp.int32, (1, _LBLK), 1)) // _SHIFT
    tags = tags_ref[...]
    acc = jnp.zeros((_B, _LBLK), jnp.bool_)
    for k in range(_K):
        acc = acc | (tags[:, k : k + 1] == tag_ids)
    mask_ref[...] = acc.astype(jnp.int32)


def kernel(tags, embedding):
    num_l = _L // _LBLK
    embed, mask = pl.pallas_call(
        _copy_mask_kernel,
        grid=(num_l,),
        in_specs=[
            pl.BlockSpec((_B, _K), lambda l: (0, 0)),
            pl.BlockSpec((_LBLK, _D), lambda l: (l, 0)),
        ],
        out_specs=[
            pl.BlockSpec((_B, _LBLK, _D), lambda l: (0, l, 0)),
            pl.BlockSpec((_B, _LBLK), lambda l: (0, l)),
        ],
        out_shape=[
            jax.ShapeDtypeStruct((_B, _L, _D), jnp.float32),
            jax.ShapeDtypeStruct((_B, _L), jnp.int32),
        ],
    )(tags.astype(jnp.int32), embedding)
    return embed, mask
